# baseline (speedup 1.0000x reference)
import functools

import jax
import jax.numpy as jnp
from jax import lax
from jax.experimental import pallas as pl
from jax.experimental.pallas import tpu as pltpu
from jax.experimental.pallas import tpu_sc as plsc

NC, NS, L = 2, 16, 16
NW = NC * NS
B, H, D = 4096, 200, 64
DW = 128
RPW = B // NW
EPW = RPW * H
CH = 64
NCHUNK = EPW // CH
NBUF = 6
LA = NBUF - 2
ACC_ROWS = NS * RPW
TRASH0 = ACC_ROWS


def _iota16():
    return lax.broadcasted_iota(jnp.int32, (L,), 0)


def _body(ids_hbm, mask_hbm, table_hbm, out_hbm,
          ids_v, mask_v, inv_v, blk_v, obuf, acc_sh, *ring):
    gbufs = ring[:NBUF]
    dsts = ring[NBUF:2 * NBUF]
    sem_g = ring[2 * NBUF:3 * NBUF]
    sem_s = ring[3 * NBUF:4 * NBUF]

    c = lax.axis_index("c")
    s = lax.axis_index("s")
    wid = c * NS + s
    ebase = wid * EPW
    row_base = wid * RPW
    slot_base = s * RPW
    trash = TRASH0 + s

    pltpu.sync_copy(ids_hbm.at[pl.ds(ebase, EPW)], ids_v)
    pltpu.sync_copy(mask_hbm.at[pl.ds(ebase, EPW)], mask_v.at[pl.ds(0, EPW)])

    zeros = jnp.zeros((L,), jnp.float32)
    for i in range(L):
        for j in range(DW // L):
            blk_v[i, pl.ds(j * L, L)] = zeros

    def zero_body(g, _):
        pltpu.sync_copy(blk_v, acc_sh.at[pl.ds(slot_base + g * L, L)])
        return 0
    lax.fori_loop(0, RPW // L, zero_body, 0)

    def inv_body(r, _):
        cntv = jnp.zeros((L,), jnp.int32)
        for k in range(H // L):
            mb = mask_v[pl.ds(r * H + k * L, L)] > 0
            cntv = cntv + plsc.all_reduce_population_count(mb)
        tail = mask_v[pl.ds(r * H + (H // L) * L, L)] > 0
        tail = jnp.logical_and(tail, _iota16() < (H % L))
        cntv = cntv + plsc.all_reduce_population_count(tail)
        cf = jnp.maximum(cntv.astype(jnp.float32), 1.0)
        inv_v[pl.ds(r * L, L)] = 1.0 / cf
        return 0
    lax.fori_loop(0, RPW, inv_body, 0)

    def gather_desc(g, b):
        return pltpu.make_async_copy(
            table_hbm.at[ids_v.at[pl.ds(g * CH, CH)]], gbufs[b], sem_g[b])

    def scatter_start(b):
        pltpu.async_copy(gbufs[b], acc_sh.at[dsts[b]], sem_s[b], add=True)

    def scatter_wait(b):
        pltpu.make_async_copy(gbufs[b], acc_sh.at[dsts[b]], sem_s[b]).wait()

    for g0 in range(LA):
        gather_desc(g0, g0).start()

    def ring_body(i, _):
        for b in range(NBUF):
            g = i * NBUF + b
            b2 = (b - 2) % NBUF

            @pl.when(g >= 2)
            def _():
                scatter_wait(b2)

            @pl.when(g + LA < NCHUNK)
            def _():
                gather_desc(g + LA, b2).start()

            off = g * CH
            db = dsts[b]
            for j in range(CH // L):
                m = mask_v[pl.ds(off + j * L, L)]
                e = off + j * L + _iota16()
                slot = slot_base + e // H
                db[pl.ds(j * L, L)] = jnp.where(m > 0, slot, trash)
            gather_desc(g, b).wait()
            scatter_start(b)
        return 0
    lax.fori_loop(0, NCHUNK // NBUF, ring_body, 0)

    for k in range(NBUF * (NCHUNK // NBUF), NCHUNK):
        b = k % NBUF
        scatter_wait((b - 2) % NBUF)
        off = k * CH
        db = dsts[b]
        for j in range(CH // L):
            m = mask_v[pl.ds(off + j * L, L)]
            e = off + j * L + _iota16()
            slot = slot_base + e // H
            db[pl.ds(j * L, L)] = jnp.where(m > 0, slot, trash)
        gather_desc(k, b).wait()
        scatter_start(b)

    scatter_wait((NCHUNK - 2) % NBUF)
    scatter_wait((NCHUNK - 1) % NBUF)

    def out_body(gb, _):
        pltpu.sync_copy(acc_sh.at[pl.ds(slot_base + gb * L, L)], blk_v)
        for i in range(L):
            inv = inv_v[pl.ds(gb * (L * L) + i * L, L)]
            for j in range(D // L):
                obuf[i, pl.ds(j * L, L)] = blk_v[i, pl.ds(j * L, L)] * inv
        pltpu.sync_copy(obuf, out_hbm.at[pl.ds(row_base + gb * L, L)])
        return 0
    lax.fori_loop(0, RPW // L, out_body, 0)


@jax.jit
def _sc_pool(ids_flat, mask_flat, table):
    mesh = plsc.VectorSubcoreMesh(core_axis_name="c", subcore_axis_name="s")
    f = pl.kernel(
        _body,
        out_type=jax.ShapeDtypeStruct((B, D), jnp.float32),
        mesh=mesh,
        compiler_params=pltpu.CompilerParams(needs_layout_passes=False,
                                             use_tc_tiling_on_sc=False),
        scratch_types=(
            [
                pltpu.VMEM((EPW,), jnp.int32),
                pltpu.VMEM((EPW + L,), jnp.int32),
                pltpu.VMEM((RPW * L,), jnp.float32),
                pltpu.VMEM((L, DW), jnp.float32),
                pltpu.VMEM((L, D), jnp.float32),
                pltpu.VMEM_SHARED((ACC_ROWS + NS, DW), jnp.float32),
            ]
            + [pltpu.VMEM((CH, DW), jnp.float32)] * NBUF
            + [pltpu.VMEM((CH,), jnp.int32)] * NBUF
            + [pltpu.SemaphoreType.DMA] * (2 * NBUF)
        ),
    )
    return f(ids_flat, mask_flat, table)


def kernel(input_ids, attention_mask, table):
    ids_flat = input_ids.reshape(-1)
    mask_flat = attention_mask.reshape(-1)
    table_p = jnp.pad(table, ((0, 0), (0, DW - D)))
    return _sc_pool(ids_flat, mask_flat, table_p)


# speedup vs baseline: 1.0483x; 1.0483x over previous
import functools

import jax
import jax.numpy as jnp
from jax import lax
from jax.experimental import pallas as pl
from jax.experimental.pallas import tpu as pltpu
from jax.experimental.pallas import tpu_sc as plsc

NC, NS, L = 2, 16, 16
NW = NC * NS
B, H, D = 4096, 200, 64
RPW = B // NW
EPW = RPW * H
CH = 128
NBUF = 8
LA = NBUF - 2
ACC_ROWS = NS * RPW
TRASH0 = ACC_ROWS


def _iota16():
    return lax.broadcasted_iota(jnp.int32, (L,), 0)


def _body(ids_hbm, mask_hbm, table_hbm, out_hbm,
          ids_v, mask_v, inv_v, blk_v, acc_sh, *ring):
    gbufs = ring[:NBUF]
    dsts = ring[NBUF:2 * NBUF]
    sem_g = ring[2 * NBUF:3 * NBUF]
    sem_s = ring[3 * NBUF:4 * NBUF]

    c = lax.axis_index("c")
    s = lax.axis_index("s")
    wid = c * NS + s
    ebase = wid * EPW
    row_base = wid * RPW
    slot_base = s * RPW
    trash = TRASH0 + s

    pltpu.sync_copy(ids_hbm.at[pl.ds(ebase, EPW)], ids_v.at[pl.ds(0, EPW)])
    pltpu.sync_copy(mask_hbm.at[pl.ds(ebase, EPW)], mask_v.at[pl.ds(0, EPW)])

    zeros = jnp.zeros((L,), jnp.float32)
    for i in range(L):
        for j in range(D // L):
            blk_v[i, pl.ds(j * L, L)] = zeros

    def zero_body(g, _):
        pltpu.sync_copy(blk_v, acc_sh.at[pl.ds(slot_base + g * L, L)])
        return 0
    lax.fori_loop(0, RPW // L, zero_body, 0)

    def inv_body(r, _):
        cntv = jnp.zeros((L,), jnp.int32)
        for k in range(H // L):
            mb = mask_v[pl.ds(r * H + k * L, L)] > 0
            cntv = cntv + plsc.all_reduce_population_count(mb)
        tail = mask_v[pl.ds(r * H + (H // L) * L, L)] > 0
        tail = jnp.logical_and(tail, _iota16() < (H % L))
        cntv = cntv + plsc.all_reduce_population_count(tail)
        cf = jnp.maximum(cntv.astype(jnp.float32), 1.0)
        inv_v[pl.ds(r * L, L)] = 1.0 / cf
        return 0
    lax.fori_loop(0, RPW, inv_body, 0)

    def comp_body(t, ptr):
        idv = ids_v[pl.ds(t * L, L)]
        m = mask_v[pl.ds(t * L, L)]
        mb = m > 0
        slotv = slot_base + (t * L + _iota16()) // H
        plsc.store_compressed(ids_v.at[pl.ds(ptr, L)], idv, mask=mb)
        plsc.store_compressed(mask_v.at[pl.ds(ptr, L)], slotv, mask=mb)
        return ptr + plsc.all_reduce_population_count(mb)[0]
    kept = lax.fori_loop(0, EPW // L, comp_body, jnp.int32(0))

    for j in range(CH // L):
        ids_v[pl.ds(kept + j * L, L)] = jnp.zeros((L,), jnp.int32)
        mask_v[pl.ds(kept + j * L, L)] = jnp.broadcast_to(trash, (L,))
    n = (kept + CH - 1) // CH

    def gather_desc(g, b):
        return pltpu.make_async_copy(
            table_hbm.at[ids_v.at[pl.ds(g * CH, CH)]], gbufs[b], sem_g[b])

    def scatter_start(b):
        pltpu.async_copy(gbufs[b], acc_sh.at[dsts[b]], sem_s[b], add=True)

    def scatter_wait(b):
        pltpu.make_async_copy(gbufs[b], acc_sh.at[dsts[b]], sem_s[b]).wait()

    for g0 in range(LA):
        @pl.when(g0 < n)
        def _():
            gather_desc(g0, g0).start()

    def ring_body(i, _):
        for b in range(NBUF):
            g = i * NBUF + b
            b2 = (b - 2) % NBUF

            @pl.when(jnp.logical_and(g >= 2, g < n))
            def _():
                scatter_wait(b2)

            @pl.when(g + LA < n)
            def _():
                gather_desc(g + LA, b2).start()

            @pl.when(g < n)
            def _():
                db = dsts[b]
                for j in range(CH // L):
                    db[pl.ds(j * L, L)] = mask_v[pl.ds(g * CH + j * L, L)]
                gather_desc(g, b).wait()
                scatter_start(b)
        return 0
    lax.fori_loop(0, (n + NBUF - 1) // NBUF, ring_body, 0)

    for b in range(NBUF):
        last = jnp.logical_and(n >= 1, lax.rem(n - 1, NBUF) == b)
        prev = jnp.logical_and(n >= 2, lax.rem(n - 2, NBUF) == b)

        @pl.when(jnp.logical_or(last, prev))
        def _():
            scatter_wait(b)

    def out_body(gb, _):
        pltpu.sync_copy(acc_sh.at[pl.ds(slot_base + gb * L, L)], blk_v)
        for i in range(L):
            inv = inv_v[pl.ds(gb * (L * L) + i * L, L)]
            for j in range(D // L):
                blk_v[i, pl.ds(j * L, L)] = blk_v[i, pl.ds(j * L, L)] * inv
        pltpu.sync_copy(blk_v, out_hbm.at[pl.ds(row_base + gb * L, L)])
        return 0
    lax.fori_loop(0, RPW // L, out_body, 0)


@jax.jit
def _sc_pool(ids_flat, mask_flat, table):
    mesh = plsc.VectorSubcoreMesh(core_axis_name="c", subcore_axis_name="s")
    f = pl.kernel(
        _body,
        out_type=jax.ShapeDtypeStruct((B, D), jnp.float32),
        mesh=mesh,
        compiler_params=pltpu.CompilerParams(needs_layout_passes=False,
                                             use_tc_tiling_on_sc=False),
        scratch_types=(
            [
                pltpu.VMEM((EPW + CH,), jnp.int32),
                pltpu.VMEM((EPW + CH,), jnp.int32),
                pltpu.VMEM((RPW * L,), jnp.float32),
                pltpu.VMEM((L, D), jnp.float32),
                pltpu.VMEM_SHARED((ACC_ROWS + NS, D), jnp.float32),
            ]
            + [pltpu.VMEM((CH, D), jnp.float32)] * NBUF
            + [pltpu.VMEM((CH,), jnp.int32)] * NBUF
            + [pltpu.SemaphoreType.DMA] * (2 * NBUF)
        ),
    )
    return f(ids_flat, mask_flat, table)


def kernel(input_ids, attention_mask, table):
    ids_flat = input_ids.reshape(-1)
    mask_flat = attention_mask.reshape(-1)
    return _sc_pool(ids_flat, mask_flat, table)


# speedup vs baseline: 1.0544x; 1.0057x over previous
import functools

import jax
import jax.numpy as jnp
from jax import lax
from jax.experimental import pallas as pl
from jax.experimental.pallas import tpu as pltpu
from jax.experimental.pallas import tpu_sc as plsc

NC, NS, L = 2, 16, 16
NW = NC * NS
B, H, D = 4096, 200, 64
RPW = B // NW
EPW = RPW * H
CH = 128
NCHUNK = EPW // CH
NBUF = 8
LA = NBUF - 2
ACC_ROWS = NS * RPW
TRASH0 = ACC_ROWS


def _iota16():
    return lax.broadcasted_iota(jnp.int32, (L,), 0)


def _body(ids_hbm, mask_hbm, table_hbm, out_hbm,
          ids_v, mask_v, inv_v, blk_v, acc_sh, *ring):
    gbufs = ring[:NBUF]
    dsts = ring[NBUF:2 * NBUF]
    sem_g = ring[2 * NBUF:3 * NBUF]
    sem_s = ring[3 * NBUF:4 * NBUF]

    c = lax.axis_index("c")
    s = lax.axis_index("s")
    wid = c * NS + s
    ebase = wid * EPW
    row_base = wid * RPW
    slot_base = s * RPW
    trash = TRASH0 + s

    pltpu.sync_copy(ids_hbm.at[pl.ds(ebase, EPW)], ids_v)
    pltpu.sync_copy(mask_hbm.at[pl.ds(ebase, EPW)], mask_v.at[pl.ds(0, EPW)])

    zeros = jnp.zeros((L,), jnp.float32)
    for i in range(L):
        for j in range(D // L):
            blk_v[i, pl.ds(j * L, L)] = zeros

    def zero_body(g, _):
        pltpu.sync_copy(blk_v, acc_sh.at[pl.ds(slot_base + g * L, L)])
        return 0
    lax.fori_loop(0, RPW // L, zero_body, 0)

    def inv_body(r, _):
        cntv = jnp.zeros((L,), jnp.int32)
        for k in range(H // L):
            mb = mask_v[pl.ds(r * H + k * L, L)] > 0
            cntv = cntv + plsc.all_reduce_population_count(mb)
        tail = mask_v[pl.ds(r * H + (H // L) * L, L)] > 0
        tail = jnp.logical_and(tail, _iota16() < (H % L))
        cntv = cntv + plsc.all_reduce_population_count(tail)
        cf = jnp.maximum(cntv.astype(jnp.float32), 1.0)
        inv_v[pl.ds(r * L, L)] = 1.0 / cf
        return 0
    lax.fori_loop(0, RPW, inv_body, 0)

    def gather_desc(g, b):
        return pltpu.make_async_copy(
            table_hbm.at[ids_v.at[pl.ds(g * CH, CH)]], gbufs[b], sem_g[b])

    def scatter_start(b):
        pltpu.async_copy(gbufs[b], acc_sh.at[dsts[b]], sem_s[b], add=True)

    def scatter_wait(b):
        pltpu.make_async_copy(gbufs[b], acc_sh.at[dsts[b]], sem_s[b]).wait()

    for g0 in range(LA):
        gather_desc(g0, g0).start()

    def ring_body(i, _):
        for b in range(NBUF):
            g = i * NBUF + b
            b2 = (b - 2) % NBUF

            @pl.when(g >= 2)
            def _():
                scatter_wait(b2)

            @pl.when(g + LA < NCHUNK)
            def _():
                gather_desc(g + LA, b2).start()

            off = g * CH
            db = dsts[b]
            for j in range(CH // L):
                m = mask_v[pl.ds(off + j * L, L)]
                e = off + j * L + _iota16()
                slot = slot_base + e // H
                db[pl.ds(j * L, L)] = jnp.where(m > 0, slot, trash)
            gather_desc(g, b).wait()
            scatter_start(b)
        return 0
    lax.fori_loop(0, NCHUNK // NBUF, ring_body, 0)

    scatter_wait((NCHUNK - 2) % NBUF)
    scatter_wait((NCHUNK - 1) % NBUF)

    def out_body(gb, _):
        pltpu.sync_copy(acc_sh.at[pl.ds(slot_base + gb * L, L)], blk_v)
        for i in range(L):
            inv = inv_v[pl.ds(gb * (L * L) + i * L, L)]
            for j in range(D // L):
                blk_v[i, pl.ds(j * L, L)] = blk_v[i, pl.ds(j * L, L)] * inv
        pltpu.sync_copy(blk_v, out_hbm.at[pl.ds(row_base + gb * L, L)])
        return 0
    lax.fori_loop(0, RPW // L, out_body, 0)


@jax.jit
def _sc_pool(ids_flat, mask_flat, table):
    mesh = plsc.VectorSubcoreMesh(core_axis_name="c", subcore_axis_name="s")
    f = pl.kernel(
        _body,
        out_type=jax.ShapeDtypeStruct((B, D), jnp.float32),
        mesh=mesh,
        compiler_params=pltpu.CompilerParams(needs_layout_passes=False,
                                             use_tc_tiling_on_sc=False),
        scratch_types=(
            [
                pltpu.VMEM((EPW,), jnp.int32),
                pltpu.VMEM((EPW + L,), jnp.int32),
                pltpu.VMEM((RPW * L,), jnp.float32),
                pltpu.VMEM((L, D), jnp.float32),
                pltpu.VMEM_SHARED((ACC_ROWS + NS, D), jnp.float32),
            ]
            + [pltpu.VMEM((CH, D), jnp.float32)] * NBUF
            + [pltpu.VMEM((CH,), jnp.int32)] * NBUF
            + [pltpu.SemaphoreType.DMA] * (2 * NBUF)
        ),
    )
    return f(ids_flat, mask_flat, table)


def kernel(input_ids, attention_mask, table):
    ids_flat = input_ids.reshape(-1)
    mask_flat = attention_mask.reshape(-1)
    return _sc_pool(ids_flat, mask_flat, table)
